# Initial kernel scaffold; baseline (speedup 1.0000x reference)
import jax
import jax.numpy as jnp
from jax.experimental import pallas as pl


def kernel(x, table, W, gamma, beta):
    raise NotImplementedError("write your pallas kernel here")


# speedup vs baseline: 5.1559x; 5.1559x over previous
import functools

import jax
import jax.numpy as jnp
from jax import lax
from jax.experimental import pallas as pl
from jax.experimental.pallas import tpu as pltpu
from jax.experimental.pallas import tpu_sc as plsc

_EPS = 1e-5
_CHUNK = 128


@functools.cache
def _build_gather(B, D):
    info = plsc.get_sparse_core_info()
    NC, NS = info.num_cores, info.num_subcores
    NW = NC * NS
    b_per_w = B // NW
    n_chunks = b_per_w // _CHUNK
    mesh = plsc.VectorSubcoreMesh(core_axis_name="c", subcore_axis_name="s")

    @functools.partial(
        pl.kernel,
        mesh=mesh,
        out_type=jax.ShapeDtypeStruct((B, D), jnp.float32),
        scratch_types=[
            pltpu.VMEM((_CHUNK,), jnp.int32),
            pltpu.VMEM((_CHUNK, D), jnp.float32),
            pltpu.SemaphoreType.DMA,
        ],
    )
    def gather_k(table_hbm, idx_hbm, out_hbm, idx_v, rows_v, sem):
        wid = lax.axis_index("s") * NC + lax.axis_index("c")
        base = wid * b_per_w
        for c in range(n_chunks):
            off = base + c * _CHUNK
            pltpu.sync_copy(idx_hbm.at[pl.ds(off, _CHUNK)], idx_v)
            pltpu.async_copy(table_hbm.at[idx_v], rows_v, sem).wait()
            pltpu.sync_copy(rows_v, out_hbm.at[pl.ds(off, _CHUNK)])

    return gather_k


_BC = 2048


def _dense_body(emb_ref, w_ref, g_ref, b_ref, out_ref):
    i = pl.program_id(0)
    n = pl.num_programs(0)
    hc = lax.dot_general(
        emb_ref[...], w_ref[...], (((1,), (1,)), ((), ())),
        preferred_element_type=jnp.float32,
        precision=lax.Precision.HIGHEST,
    )
    out_ref[pl.ds(i * _BC, _BC), :] = jnp.maximum(hc, 0.0)

    @pl.when(i == n - 1)
    def _():
        h = out_ref[...]
        mean = jnp.mean(h, axis=0, keepdims=True)
        cent = h - mean
        var = jnp.mean(cent * cent, axis=0, keepdims=True)
        out_ref[...] = g_ref[...] * cent * lax.rsqrt(var + _EPS) + b_ref[...]


def kernel(x, table, W, gamma, beta):
    B = x.shape[0]
    D = table.shape[1]
    DOUT = W.shape[0]
    emb = _build_gather(B, D)(table, x.astype(jnp.int32))
    out = pl.pallas_call(
        _dense_body,
        grid=(B // _BC,),
        in_specs=[
            pl.BlockSpec((_BC, D), lambda i: (i, 0)),
            pl.BlockSpec((DOUT, D), lambda i: (0, 0)),
            pl.BlockSpec((1, DOUT), lambda i: (0, 0)),
            pl.BlockSpec((1, DOUT), lambda i: (0, 0)),
        ],
        out_specs=pl.BlockSpec((B, DOUT), lambda i: (0, 0)),
        out_shape=jax.ShapeDtypeStruct((B, DOUT), jnp.float32),
    )(emb, W, gamma.reshape(1, -1), beta.reshape(1, -1))
    return out


# speedup vs baseline: 5.8683x; 1.1382x over previous
import functools

import jax
import jax.numpy as jnp
from jax import lax
from jax.experimental import pallas as pl
from jax.experimental.pallas import tpu as pltpu
from jax.experimental.pallas import tpu_sc as plsc

_EPS = 1e-5
_CHUNK = 128


@functools.cache
def _build_gather(B, D):
    info = plsc.get_sparse_core_info()
    NC, NS = info.num_cores, info.num_subcores
    NW = NC * NS
    b_per_w = B // NW
    n_chunks = b_per_w // _CHUNK
    mesh = plsc.VectorSubcoreMesh(core_axis_name="c", subcore_axis_name="s")

    @functools.partial(
        pl.kernel,
        mesh=mesh,
        out_type=jax.ShapeDtypeStruct((B, D), jnp.float32),
        scratch_types=[
            pltpu.VMEM((_CHUNK,), jnp.int32),
            pltpu.VMEM((_CHUNK, D), jnp.float32),
            pltpu.SemaphoreType.DMA,
        ],
    )
    def gather_k(table_hbm, idx_hbm, out_hbm, idx_v, rows_v, sem):
        wid = lax.axis_index("s") * NC + lax.axis_index("c")
        base = wid * b_per_w
        for c in range(n_chunks):
            off = base + c * _CHUNK
            pltpu.sync_copy(idx_hbm.at[pl.ds(off, _CHUNK)], idx_v)
            pltpu.async_copy(table_hbm.at[idx_v], rows_v, sem).wait()
            pltpu.sync_copy(rows_v, out_hbm.at[pl.ds(off, _CHUNK)])

    return gather_k


_BC = 2048


def _dense_body(emb_ref, w_ref, g_ref, b_ref, out_ref):
    i = pl.program_id(0)
    n = pl.num_programs(0)
    hc = lax.dot_general(
        emb_ref[...], w_ref[...], (((1,), (1,)), ((), ())),
        preferred_element_type=jnp.float32,
    )
    out_ref[pl.ds(i * _BC, _BC), :] = jnp.maximum(hc, 0.0)

    @pl.when(i == n - 1)
    def _():
        h = out_ref[...]
        mean = jnp.mean(h, axis=0, keepdims=True)
        cent = h - mean
        var = jnp.mean(cent * cent, axis=0, keepdims=True)
        out_ref[...] = g_ref[...] * cent * lax.rsqrt(var + _EPS) + b_ref[...]


def kernel(x, table, W, gamma, beta):
    B = x.shape[0]
    D = table.shape[1]
    DOUT = W.shape[0]
    emb = _build_gather(B, D)(table, x.astype(jnp.int32))
    out = pl.pallas_call(
        _dense_body,
        grid=(B // _BC,),
        in_specs=[
            pl.BlockSpec((_BC, D), lambda i: (i, 0)),
            pl.BlockSpec((DOUT, D), lambda i: (0, 0)),
            pl.BlockSpec((1, DOUT), lambda i: (0, 0)),
            pl.BlockSpec((1, DOUT), lambda i: (0, 0)),
        ],
        out_specs=pl.BlockSpec((B, DOUT), lambda i: (0, 0)),
        out_shape=jax.ShapeDtypeStruct((B, DOUT), jnp.float32),
    )(emb, W, gamma.reshape(1, -1), beta.reshape(1, -1))
    return out


# speedup vs baseline: 5.8786x; 1.0017x over previous
import functools

import jax
import jax.numpy as jnp
from jax import lax
from jax.experimental import pallas as pl
from jax.experimental.pallas import tpu as pltpu
from jax.experimental.pallas import tpu_sc as plsc

_EPS = 1e-5
_CHUNK = 128


@functools.cache
def _build_gather(B, D):
    info = plsc.get_sparse_core_info()
    NC, NS = info.num_cores, info.num_subcores
    NW = NC * NS
    b_per_w = B // NW
    n_chunks = b_per_w // _CHUNK
    mesh = plsc.VectorSubcoreMesh(core_axis_name="c", subcore_axis_name="s")

    @functools.partial(
        pl.kernel,
        mesh=mesh,
        out_type=jax.ShapeDtypeStruct((B, D), jnp.float32),
        scratch_types=[
            pltpu.VMEM((_CHUNK,), jnp.int32),
            pltpu.VMEM((_CHUNK, D), jnp.float32),
            pltpu.SemaphoreType.DMA,
        ],
    )
    def gather_k(table_hbm, idx_hbm, out_hbm, idx_v, rows_v, sem):
        wid = lax.axis_index("s") * NC + lax.axis_index("c")
        base = wid * b_per_w
        for c in range(n_chunks):
            off = base + c * _CHUNK
            pltpu.sync_copy(idx_hbm.at[pl.ds(off, _CHUNK)], idx_v)
            pltpu.async_copy(table_hbm.at[idx_v], rows_v, sem).wait()
            pltpu.sync_copy(rows_v, out_hbm.at[pl.ds(off, _CHUNK)])

    return gather_k


_BC = 2048


def _dense_body(emb_ref, w_ref, g_ref, b_ref, out_ref):
    i = pl.program_id(0)
    n = pl.num_programs(0)
    hc = lax.dot_general(
        emb_ref[...].astype(jnp.bfloat16), w_ref[...].astype(jnp.bfloat16),
        (((1,), (1,)), ((), ())),
        preferred_element_type=jnp.float32,
    )
    out_ref[pl.ds(i * _BC, _BC), :] = jnp.maximum(hc, 0.0)

    @pl.when(i == n - 1)
    def _():
        h = out_ref[...]
        mean = jnp.mean(h, axis=0, keepdims=True)
        cent = h - mean
        var = jnp.mean(cent * cent, axis=0, keepdims=True)
        out_ref[...] = g_ref[...] * cent * lax.rsqrt(var + _EPS) + b_ref[...]


def kernel(x, table, W, gamma, beta):
    B = x.shape[0]
    D = table.shape[1]
    DOUT = W.shape[0]
    emb = _build_gather(B, D)(table, x.astype(jnp.int32))
    out = pl.pallas_call(
        _dense_body,
        grid=(B // _BC,),
        in_specs=[
            pl.BlockSpec((_BC, D), lambda i: (i, 0)),
            pl.BlockSpec((DOUT, D), lambda i: (0, 0)),
            pl.BlockSpec((1, DOUT), lambda i: (0, 0)),
            pl.BlockSpec((1, DOUT), lambda i: (0, 0)),
        ],
        out_specs=pl.BlockSpec((B, DOUT), lambda i: (0, 0)),
        out_shape=jax.ShapeDtypeStruct((B, DOUT), jnp.float32),
    )(emb, W, gamma.reshape(1, -1), beta.reshape(1, -1))
    return out


# speedup vs baseline: 6.1042x; 1.0384x over previous
import functools

import jax
import jax.numpy as jnp
from jax import lax
from jax.experimental import pallas as pl
from jax.experimental.pallas import tpu as pltpu
from jax.experimental.pallas import tpu_sc as plsc

_EPS = 1e-5
_CHUNK = 128


@functools.cache
def _build_gather(B, D):
    info = plsc.get_sparse_core_info()
    NC, NS = info.num_cores, info.num_subcores
    NW = NC * NS
    b_per_w = B // NW
    n_chunks = b_per_w // _CHUNK
    mesh = plsc.VectorSubcoreMesh(core_axis_name="c", subcore_axis_name="s")

    @functools.partial(
        pl.kernel,
        mesh=mesh,
        out_type=jax.ShapeDtypeStruct((B, D), jnp.float32),
        scratch_types=[
            pltpu.VMEM((b_per_w,), jnp.int32),
            pltpu.VMEM((2, _CHUNK, D), jnp.float32),
            pltpu.SemaphoreType.DMA,
            pltpu.SemaphoreType.DMA,
        ],
    )
    def gather_k(table_hbm, idx_hbm, out_hbm, idx_v, rows_v, gsem, wsem):
        wid = lax.axis_index("s") * NC + lax.axis_index("c")
        base = wid * b_per_w
        pltpu.sync_copy(idx_hbm.at[pl.ds(base, b_per_w)], idx_v)
        prev_write = None
        for c in range(n_chunks):
            g = pltpu.async_copy(
                table_hbm.at[idx_v.at[pl.ds(c * _CHUNK, _CHUNK)]],
                rows_v.at[c % 2], gsem)
            if prev_write is not None:
                prev_write.wait()
            g.wait()
            prev_write = pltpu.async_copy(
                rows_v.at[c % 2], out_hbm.at[pl.ds(base + c * _CHUNK, _CHUNK)],
                wsem)
        prev_write.wait()

    return gather_k


_BC = 2048


def _dense_body(emb_ref, w_ref, g_ref, b_ref, out_ref):
    i = pl.program_id(0)
    n = pl.num_programs(0)
    hc = lax.dot_general(
        emb_ref[...], w_ref[...], (((1,), (1,)), ((), ())),
        preferred_element_type=jnp.float32,
    )
    out_ref[pl.ds(i * _BC, _BC), :] = jnp.maximum(hc, 0.0)

    @pl.when(i == n - 1)
    def _():
        h = out_ref[...]
        mean = jnp.mean(h, axis=0, keepdims=True)
        cent = h - mean
        var = jnp.mean(cent * cent, axis=0, keepdims=True)
        out_ref[...] = g_ref[...] * cent * lax.rsqrt(var + _EPS) + b_ref[...]


def kernel(x, table, W, gamma, beta):
    B = x.shape[0]
    D = table.shape[1]
    DOUT = W.shape[0]
    emb = _build_gather(B, D)(table, x.astype(jnp.int32))
    out = pl.pallas_call(
        _dense_body,
        grid=(B // _BC,),
        in_specs=[
            pl.BlockSpec((_BC, D), lambda i: (i, 0)),
            pl.BlockSpec((DOUT, D), lambda i: (0, 0)),
            pl.BlockSpec((1, DOUT), lambda i: (0, 0)),
            pl.BlockSpec((1, DOUT), lambda i: (0, 0)),
        ],
        out_specs=pl.BlockSpec((B, DOUT), lambda i: (0, 0)),
        out_shape=jax.ShapeDtypeStruct((B, DOUT), jnp.float32),
    )(emb, W, gamma.reshape(1, -1), beta.reshape(1, -1))
    return out


# speedup vs baseline: 6.4395x; 1.0549x over previous
import functools

import jax
import jax.numpy as jnp
from jax import lax
from jax.experimental import pallas as pl
from jax.experimental.pallas import tpu as pltpu
from jax.experimental.pallas import tpu_sc as plsc

_EPS = 1e-5
_CHUNK = 128


@functools.cache
def _build_gather(B, D):
    info = plsc.get_sparse_core_info()
    NC, NS = info.num_cores, info.num_subcores
    NW = NC * NS
    b_per_w = B // NW
    n_chunks = b_per_w // _CHUNK
    mesh = plsc.VectorSubcoreMesh(core_axis_name="c", subcore_axis_name="s")

    @functools.partial(
        pl.kernel,
        mesh=mesh,
        out_type=jax.ShapeDtypeStruct((B, D), jnp.float32),
        scratch_types=[
            pltpu.VMEM((b_per_w,), jnp.int32),
            pltpu.VMEM((2, _CHUNK, D), jnp.float32),
            pltpu.SemaphoreType.DMA,
            pltpu.SemaphoreType.DMA,
        ],
    )
    def gather_k(table_hbm, idx_hbm, out_hbm, idx_v, rows_v, gsem, wsem):
        wid = lax.axis_index("s") * NC + lax.axis_index("c")
        base = wid * b_per_w
        pltpu.sync_copy(idx_hbm.at[pl.ds(base, b_per_w)], idx_v)
        prev_write = None
        for c in range(n_chunks):
            g = pltpu.async_copy(
                table_hbm.at[idx_v.at[pl.ds(c * _CHUNK, _CHUNK)]],
                rows_v.at[c % 2], gsem)
            if prev_write is not None:
                prev_write.wait()
            g.wait()
            prev_write = pltpu.async_copy(
                rows_v.at[c % 2], out_hbm.at[pl.ds(base + c * _CHUNK, _CHUNK)],
                wsem)
        prev_write.wait()

    return gather_k


_BC = 2048
_NB = 16384 // _BC


def _dense_body(emb_ref, w_ref, g_ref, b_ref, out_ref, h_ref, stats_ref):
    p = pl.program_id(0)
    i = pl.program_id(1)

    @pl.when(p == 0)
    def _matmul_phase():
        hc = lax.dot_general(
            emb_ref[...], w_ref[...], (((1,), (1,)), ((), ())),
            preferred_element_type=jnp.float32,
        )
        hc = jnp.maximum(hc, 0.0)
        h_ref[pl.ds(i * _BC, _BC), :] = hc
        part = jnp.concatenate(
            [jnp.sum(hc, axis=0, keepdims=True),
             jnp.sum(hc * hc, axis=0, keepdims=True)], axis=0)

        @pl.when(i == 0)
        def _():
            stats_ref[...] = part

        @pl.when(i > 0)
        def _():
            stats_ref[...] = stats_ref[...] + part

    @pl.when(p == 1)
    def _normalize_phase():
        n_rows = _BC * _NB
        stats = stats_ref[...]
        mean = stats[0:1, :] * (1.0 / n_rows)
        var = stats[1:2, :] * (1.0 / n_rows) - mean * mean
        scale = g_ref[...] * lax.rsqrt(var + _EPS)
        hc = h_ref[pl.ds(i * _BC, _BC), :]
        out_ref[...] = scale * (hc - mean) + b_ref[...]


def kernel(x, table, W, gamma, beta):
    B = x.shape[0]
    D = table.shape[1]
    DOUT = W.shape[0]
    emb = _build_gather(B, D)(table, x.astype(jnp.int32))
    last = B // _BC - 1
    out = pl.pallas_call(
        _dense_body,
        grid=(2, B // _BC),
        in_specs=[
            pl.BlockSpec((_BC, D),
                         lambda p, i: (jnp.where(p == 0, i, last), 0)),
            pl.BlockSpec((DOUT, D), lambda p, i: (0, 0)),
            pl.BlockSpec((1, DOUT), lambda p, i: (0, 0)),
            pl.BlockSpec((1, DOUT), lambda p, i: (0, 0)),
        ],
        out_specs=pl.BlockSpec((_BC, DOUT),
                               lambda p, i: (jnp.where(p == 0, 0, i), 0)),
        out_shape=jax.ShapeDtypeStruct((B, DOUT), jnp.float32),
        scratch_shapes=[
            pltpu.VMEM((B, DOUT), jnp.float32),
            pltpu.VMEM((2, DOUT), jnp.float32),
        ],
    )(emb, W, gamma.reshape(1, -1), beta.reshape(1, -1))
    return out
